# baseline (speedup 1.0000x reference)
import functools

import jax
import jax.numpy as jnp
import numpy as np
from jax import lax
from jax.experimental import pallas as pl
from jax.experimental.pallas import tpu as pltpu
from jax.experimental.pallas import tpu_sc as plsc

B = 4096
N = 64
NN = N * N
OUT = N * (N + 1) // 2

_info = plsc.get_sparse_core_info()
_NC = _info.num_cores
_NS = _info.num_subcores
_NW = _NC * _NS
_CHUNK = B // _NW
_G = 8
_STEPS = _CHUNK // _G
_NV = _G * OUT // 16


def _triu_flat() -> np.ndarray:
    rows, cols = np.triu_indices(N)
    return (rows * N + cols).astype(np.int32)


def _idx_table() -> np.ndarray:
    t = _triu_flat()
    return (np.arange(_G, dtype=np.int32)[:, None] * NN + t[None, :]).reshape(-1)


def _body(in_hbm, idx_hbm, out_hbm, vidx, vin, vout):
    wid = lax.axis_index("s") * _NC + lax.axis_index("c")
    in_base = wid * (_CHUNK * NN)
    out_base = wid * (_CHUNK * OUT)
    pltpu.sync_copy(idx_hbm, vidx)
    for step in range(_STEPS):
        pltpu.sync_copy(in_hbm.at[pl.ds(in_base + step * (_G * NN), _G * NN)],
                        vin)

        @plsc.parallel_loop(0, _NV, 1, unroll=8)
        def _gather(j):
            off = pl.multiple_of(j * 16, 16)
            iv = vidx[pl.ds(off, 16)]
            vout[pl.ds(off, 16)] = plsc.load_gather(vin, [iv])

        pltpu.sync_copy(vout,
                        out_hbm.at[pl.ds(out_base + step * (_G * OUT), _G * OUT)])


def kernel(input):
    mesh = plsc.VectorSubcoreMesh(core_axis_name="c", subcore_axis_name="s")
    k = functools.partial(
        pl.kernel,
        out_type=jax.ShapeDtypeStruct((B * OUT,), jnp.float32),
        mesh=mesh,
        scratch_types=[
            pltpu.VMEM((_G * OUT,), jnp.int32),
            pltpu.VMEM((_G * NN,), jnp.float32),
            pltpu.VMEM((_G * OUT,), jnp.float32),
        ],
        compiler_params=pltpu.CompilerParams(use_tc_tiling_on_sc=False,
                                             needs_layout_passes=False),
    )(_body)
    flat = k(input.reshape(B * NN), jnp.asarray(_idx_table()))
    return flat.reshape(B, OUT)
